# Initial kernel scaffold; baseline (speedup 1.0000x reference)
import jax
import jax.numpy as jnp
from jax.experimental import pallas as pl


def kernel(h1, e1, h2, e2, params, edge_index1, graph_ids1, mask1, edge_index2, graph_ids2, mask2):
    raise NotImplementedError("write your pallas kernel here")


# speedup vs baseline: 10.6526x; 10.6526x over previous
import functools

import jax
import jax.numpy as jnp
from jax import lax
from jax.experimental import pallas as pl
from jax.experimental.pallas import tpu as pltpu
from jax.experimental.pallas import tpu_sc as plsc

N = 10000
E = 320000
B = 16
NODE_IN = 133
EDGE_IN = 14
G = 64

NC = 2
NS = 16
NW = NC * NS
EPW = E // NW
BE = 80
NB = EPW // BE
PC = 80
NPAD = 10240
RPW = NPAD // NS
BN = 2000
NEG_SLOPE = 0.01

_f32 = jnp.float32


def _leaky(x):
    return jnp.maximum(x, NEG_SLOPE * x)


def _sc_exp(x):
    x = jnp.clip(x, -80.0, 80.0)
    t = x * 1.4426950408889634
    n = (t + jnp.where(t >= 0, 0.5, -0.5)).astype(jnp.int32)
    nf_ = n.astype(_f32)
    r = (x - nf_ * 0.693359375) + nf_ * 2.1219444005469057e-4
    p = 1.0 + r * (1.0 + r * (0.5 + r * (0.16666666666666666 + r * (
        0.041666666666666664 + r * (0.008333333333333333 + r * 0.001388888888888889)))))
    scale = lax.bitcast_convert_type((n + 127) << 23, _f32)
    return p * scale


def _bf16_round(x):
    i = lax.bitcast_convert_type(x, jnp.int32)
    rounded = i + 0x8000 + ((i >> 16) & 1)
    return lax.bitcast_convert_type(rounded & ~0xFFFF, _f32)


def _bcast16(v, j):
    dnums = lax.GatherDimensionNumbers(
        offset_dims=(), collapsed_slice_dims=(0,), start_index_map=(0,))
    idx = jnp.full((16, 1), j, jnp.int32)
    return lax.gather(v, idx, dnums, (1,),
                      mode=lax.GatherScatterMode.PROMISE_IN_BOUNDS)


def _mesh():
    return plsc.VectorSubcoreMesh(
        core_axis_name="c", subcore_axis_name="s", num_cores=NC, num_subcores=NS)


def _worker_prologue(zrow, acc):
    cid = lax.axis_index("c")
    sid = lax.axis_index("s")
    wid = sid * NC + cid
    for r in range(BE):
        for c in range(PC // 16):
            zrow[r, pl.ds(c * 16, 16)] = jnp.zeros((16,), _f32)
    r0 = sid * RPW
    for k in range(RPW // BE):
        pltpu.sync_copy(zrow, acc.at[pl.ds(r0 + k * BE, BE)])
    plsc.subcore_barrier()
    return wid, r0


def _worker_epilogue(acc, out, r0):
    plsc.subcore_barrier()
    cid = lax.axis_index("c")
    pltpu.sync_copy(acc.at[pl.ds(r0, RPW)], out.at[cid, pl.ds(r0, RPW)])


def _sweep_a_body(hp_hbm, ep_hbm, dsc_hbm, wb_hbm, src_hbm, dst_hbm, out_hbm,
                  si, di, rows, epv, packed, dscv, wbv, zrow, acc, sem):
    wid, r0 = _worker_prologue(zrow, acc)
    pltpu.sync_copy(dsc_hbm, dscv)
    pltpu.sync_copy(wb_hbm, wbv)
    lane = lax.broadcasted_iota(jnp.int32, (16,), 0)

    def body(b, carry):
        base = wid * EPW + b * BE
        pltpu.sync_copy(src_hbm.at[pl.ds(base, BE)], si.at[0])
        pltpu.sync_copy(dst_hbm.at[pl.ds(base, BE)], di.at[0])
        pltpu.async_copy(hp_hbm.at[si.at[0]], rows, sem).wait()
        pltpu.sync_copy(ep_hbm.at[pl.ds(base, BE)], epv)
        for g in range(BE // 16):
            dstg = di[0, pl.ds(g * 16, 16)]
            dscg = plsc.load_gather(dscv, [dstg])
            for j in range(16):
                e = g * 16 + j
                he = []
                acc_dot = None
                for c in range(G // 16):
                    x = rows[e, pl.ds(c * 16, 16)] + epv[e, pl.ds(c * 16, 16)]
                    h = _bf16_round(_leaky(x))
                    he.append(h)
                    term = h * wbv[pl.ds(c * 16, 16)]
                    acc_dot = term if acc_dot is None else acc_dot + term
                t = jnp.sum(acc_dot)
                lv = _leaky(_bcast16(dscg, j) + t)
                exv = _sc_exp(lv)
                for c in range(G // 16):
                    packed[e, pl.ds(c * 16, 16)] = he[c] * exv
                packed[e, pl.ds(G, 16)] = jnp.where(lane == 0, exv, 0.0)
        pltpu.sync_copy(packed, acc.at[di.at[0]], add=True)
        return carry

    lax.fori_loop(0, NB, body, 0)
    _worker_epilogue(acc, out_hbm, r0)


def _sweep_b_body(nf_hbm, sd_hbm, ss_hbm, src_hbm, dst_hbm, out_hbm,
                  si, di, rows, packed, sdv, ssv, zrow, acc, sem):
    wid, r0 = _worker_prologue(zrow, acc)
    pltpu.sync_copy(sd_hbm, sdv)
    pltpu.sync_copy(ss_hbm, ssv)
    lane = lax.broadcasted_iota(jnp.int32, (16,), 0)

    def body(b, carry):
        base = wid * EPW + b * BE
        pltpu.sync_copy(src_hbm.at[pl.ds(base, BE)], si.at[0])
        pltpu.sync_copy(dst_hbm.at[pl.ds(base, BE)], di.at[0])
        pltpu.async_copy(nf_hbm.at[si.at[0]], rows, sem).wait()
        for g in range(BE // 16):
            srcg = si[0, pl.ds(g * 16, 16)]
            dstg = di[0, pl.ds(g * 16, 16)]
            sdg = plsc.load_gather(sdv, [dstg])
            ssg = plsc.load_gather(ssv, [srcg])
            exg = _sc_exp(_leaky(sdg + ssg))
            for j in range(16):
                e = g * 16 + j
                exv = _bcast16(exg, j)
                for c in range(G // 16):
                    packed[e, pl.ds(c * 16, 16)] = (
                        _bf16_round(rows[e, pl.ds(c * 16, 16)]) * exv)
                packed[e, pl.ds(G, 16)] = jnp.where(lane == 0, exv, 0.0)
        pltpu.sync_copy(packed, acc.at[di.at[0]], add=True)
        return carry

    lax.fori_loop(0, NB, body, 0)
    _worker_epilogue(acc, out_hbm, r0)


_sweep_cache = {}


def _get_sweep(kind):
    if kind not in _sweep_cache:
        common = [
            pltpu.VMEM((1, BE), jnp.int32),
            pltpu.VMEM((1, BE), jnp.int32),
            pltpu.VMEM((BE, G), _f32),
        ]
        if kind == 'a':
            body = _sweep_a_body
            scratch = common + [
                pltpu.VMEM((BE, G), _f32),
                pltpu.VMEM((BE, PC), _f32),
                pltpu.VMEM((N,), _f32),
                pltpu.VMEM((G,), _f32),
            ]
        else:
            body = _sweep_b_body
            scratch = common + [
                pltpu.VMEM((BE, PC), _f32),
                pltpu.VMEM((N,), _f32),
                pltpu.VMEM((N,), _f32),
            ]
        scratch += [
            pltpu.VMEM((BE, PC), _f32),
            pltpu.VMEM_SHARED((NPAD, PC), _f32),
            pltpu.SemaphoreType.DMA,
        ]
        _sweep_cache[kind] = pl.kernel(
            body,
            out_type=jax.ShapeDtypeStruct((NC, NPAD, PC), _f32),
            mesh=_mesh(),
            scratch_types=scratch,
            compiler_params=pltpu.CompilerParams(
                needs_layout_passes=False, use_tc_tiling_on_sc=False),
        )
    return _sweep_cache[kind]


def _sweep_a(*args):
    return _get_sweep('a')(*args)


def _sweep_b(*args):
    return _get_sweep('b')(*args)


def _node_proj_body(h_ref, wnt, bn_, wht, wa, b2, hv_o, hp_o, dsc_o):
    h = h_ref[...]
    hv = _leaky(jnp.dot(h, wnt[...], preferred_element_type=_f32) + bn_[...])
    hv_o[...] = hv
    hp_o[...] = jnp.dot(h, wht[...], preferred_element_type=_f32)
    d = jnp.dot(hv, wa[...], preferred_element_type=_f32) + b2[...]
    col = lax.broadcasted_iota(jnp.int32, (d.shape[0], 8), 1)
    dsc_o[...] = jnp.where(col == 0, d, 0.0)


def _node_proj(h, wnt, bn_, wht, wa, b2):
    grid = N // BN
    return pl.pallas_call(
        _node_proj_body,
        grid=(grid,),
        in_specs=[
            pl.BlockSpec((BN, NODE_IN), lambda i: (i, 0)),
            pl.BlockSpec((NODE_IN, G), lambda i: (0, 0)),
            pl.BlockSpec((1, G), lambda i: (0, 0)),
            pl.BlockSpec((NODE_IN, G), lambda i: (0, 0)),
            pl.BlockSpec((G, 1), lambda i: (0, 0)),
            pl.BlockSpec((1, 1), lambda i: (0, 0)),
        ],
        out_specs=[
            pl.BlockSpec((BN, G), lambda i: (i, 0)),
            pl.BlockSpec((BN, G), lambda i: (i, 0)),
            pl.BlockSpec((BN, 8), lambda i: (i, 0)),
        ],
        out_shape=[
            jax.ShapeDtypeStruct((N, G), _f32),
            jax.ShapeDtypeStruct((N, G), _f32),
            jax.ShapeDtypeStruct((N, 8), _f32),
        ],
    )(h, wnt, bn_, wht, wa, b2)


def _eproj_body(e_ref, wet, b1, o_ref):
    o_ref[...] = jnp.dot(e_ref[...], wet[...], preferred_element_type=_f32) + b1[...]


def _eproj(e, wet, b1):
    BEJ = 8000
    return pl.pallas_call(
        _eproj_body,
        grid=(E // BEJ,),
        in_specs=[
            pl.BlockSpec((BEJ, EDGE_IN), lambda i: (i, 0)),
            pl.BlockSpec((EDGE_IN, G), lambda i: (0, 0)),
            pl.BlockSpec((1, G), lambda i: (0, 0)),
        ],
        out_specs=pl.BlockSpec((BEJ, G), lambda i: (i, 0)),
        out_shape=jax.ShapeDtypeStruct((E, G), _f32),
    )(e, wet, b1)


def _gru_block(x, h, wit, bi, wht, bh):
    gi = jnp.dot(x, wit, preferred_element_type=_f32) + bi
    gh = jnp.dot(h, wht, preferred_element_type=_f32) + bh
    r = jax.nn.sigmoid(gi[:, :G] + gh[:, :G])
    z = jax.nn.sigmoid(gi[:, G:2 * G] + gh[:, G:2 * G])
    n = jnp.tanh(gi[:, 2 * G:] + r * gh[:, 2 * G:])
    return (1.0 - z) * n + z * h


def _fuse_body(has_prep, acc_ref, prev_ref, wct, bc, wit, bi, wht, bh,
               wpt, bp, nf_o, prep_o=None):
    a0 = acc_ref[0]
    a1 = acc_ref[1]
    num = a0[:, :G] + a1[:, :G]
    s = a0[:, G:G + 1] + a1[:, G:G + 1]
    nz = s > 0.0
    sm = jnp.where(nz, num / jnp.where(nz, s, 1.0), 0.0)
    smh = sm.astype(jnp.bfloat16).astype(_f32)
    sml = sm - smh
    wcf = wct[...]
    c = (jnp.dot(smh, wcf, preferred_element_type=_f32)
         + jnp.dot(sml, wcf, preferred_element_type=_f32)
         + jnp.where(nz, 1.0, 0.0) * bc[...])
    x = jnp.where(c > 0, c, jnp.exp(jnp.minimum(c, 0.0)) - 1.0)
    prev = prev_ref[...]
    nf = jnp.maximum(_gru_block(x, prev, wit[...], bi[...], wht[...], bh[...]), 0.0)
    nf_o[...] = nf
    if has_prep:
        prep_o[...] = jnp.dot(nf, wpt[...], preferred_element_type=_f32) + bp[...]


def _fuse(acc, prev, wct, bc, wit, bi, wht, bh, wpt, bp, has_prep):
    grid = N // BN
    in_specs = [
        pl.BlockSpec((NC, BN, PC), lambda i: (0, i, 0)),
        pl.BlockSpec((BN, G), lambda i: (i, 0)),
        pl.BlockSpec((G, G), lambda i: (0, 0)),
        pl.BlockSpec((1, G), lambda i: (0, 0)),
        pl.BlockSpec((G, 3 * G), lambda i: (0, 0)),
        pl.BlockSpec((1, 3 * G), lambda i: (0, 0)),
        pl.BlockSpec((G, 3 * G), lambda i: (0, 0)),
        pl.BlockSpec((1, 3 * G), lambda i: (0, 0)),
        pl.BlockSpec((G, 8), lambda i: (0, 0)),
        pl.BlockSpec((1, 8), lambda i: (0, 0)),
    ]
    out_specs = [pl.BlockSpec((BN, G), lambda i: (i, 0))]
    out_shape = [jax.ShapeDtypeStruct((N, G), _f32)]
    if has_prep:
        out_specs.append(pl.BlockSpec((BN, 8), lambda i: (i, 0)))
        out_shape.append(jax.ShapeDtypeStruct((N, 8), _f32))
    res = pl.pallas_call(
        functools.partial(_fuse_body, has_prep),
        grid=(grid,),
        in_specs=in_specs,
        out_specs=out_specs,
        out_shape=out_shape,
    )(acc, prev, wct, bc, wit, bi, wht, bh, wpt, bp)
    return res if has_prep else (res[0], None)


def _readout_body(p, nf1, nf2, gid1, gid2, w1, w2, out_ref):
    def one(nf_ref, gid_ref, w_ref):
        nf = nf_ref[...]
        w = w_ref[...]
        msk = w > 0.5
        g = gid_ref[...]

        oh = (g == lax.broadcasted_iota(jnp.int32, (N, B), 1)).astype(_f32)

        def seg_sum(x):
            xh = x.astype(jnp.bfloat16).astype(_f32)
            xl = x - xh
            dn = (((0,), (0,)), ((), ()))
            return (lax.dot_general(oh, xh, dn, preferred_element_type=_f32)
                    + lax.dot_general(oh, xl, dn, preferred_element_type=_f32))

        def seg_max(x):
            zb = jnp.where(oh > 0.5, x, -1e30)
            return jnp.max(zb, axis=0, keepdims=True).T

        def expand(v):
            vh = v.astype(jnp.bfloat16).astype(_f32)
            vl = v - vh
            return (jnp.dot(oh, vh, preferred_element_type=_f32)
                    + jnp.dot(oh, vl, preferred_element_type=_f32))

        gf = seg_sum(nf * w)
        for t in range(2):
            wcl, bcl = p['ro%d_cl' % t]
            gfr = jnp.maximum(gf, 0.0)
            q1 = jnp.dot(gfr, wcl[:G, :], preferred_element_type=_f32)
            q2 = jnp.dot(nf, wcl[G:, :], preferred_element_type=_f32)
            z = _leaky(expand(q1) + q2 + bcl)
            z = jnp.where(msk, z, -1e30)
            mn = expand(seg_max(z))
            ex = jnp.exp(z - mn) * w
            sn = expand(seg_sum(ex))
            a = ex / (sn + 1e-12)
            wpn, bpn = p['ro%d_pn' % t]
            hv = jnp.dot(nf, wpn, preferred_element_type=_f32) + bpn
            gr = seg_sum(a * hv)
            grr = jnp.where(gr > 0, gr, jnp.exp(jnp.minimum(gr, 0.0)) - 1.0)
            wit, bi, wht, bh = p['ro%d_gru' % t]
            gf = _gru_block(jnp.maximum(grr, 0.0), gf, wit, bi, wht, bh)
        return gf

    hs1 = one(nf1, gid1, w1)
    hs2 = one(nf2, gid2, w2)
    x = jnp.concatenate([hs1, hs2, hs1 - hs2], axis=1)
    w0t, b0 = p['ffn0']
    x = jnp.maximum(jnp.dot(x, w0t, preferred_element_type=_f32) + b0, 0.0)
    w1t, b1_ = p['ffn1']
    x = jnp.maximum(jnp.dot(x, w1t, preferred_element_type=_f32) + b1_, 0.0)
    wot, bo = p['ffn_out']
    out_ref[...] = jnp.dot(x, wot, preferred_element_type=_f32) + bo


def _readout_ffn(ro_params, nf1, nf2, gid1, gid2, w1, w2):
    flat, treedef = jax.tree.flatten(ro_params)

    def body(nf1_r, nf2_r, gid1_r, gid2_r, w1_r, w2_r, *rest):
        out_ref = rest[-1]
        p_refs = jax.tree.unflatten(treedef, [r[...] for r in rest[:-1]])
        _readout_body(p_refs, nf1_r, nf2_r, gid1_r, gid2_r, w1_r, w2_r, out_ref)

    return pl.pallas_call(
        body,
        out_shape=jax.ShapeDtypeStruct((B, 1), _f32),
    )(nf1, nf2, gid1, gid2, w1, w2, *flat)


def _bfw(x):
    return x.astype(jnp.bfloat16).astype(_f32)


def _encode(p, h, e, src, dst):
    wnt = p['gc_pn'][0].T
    bn_ = p['gc_pn'][1][None, :]
    w1, b1 = p['gc_pe1']
    wht = w1[:, :NODE_IN].T
    wet = w1[:, NODE_IN:].T
    w2, b2 = p['gc_pe2']
    wa = w2[0, :G][:, None]
    wb = w2[0, G:]
    b2s = b2.reshape(1, 1)

    hv, hp, dsc8 = _node_proj(h, wnt, bn_, wht, wa, b2s)
    ep = _eproj(e, wet, b1[None, :])
    dsc = dsc8[:, 0]

    def prep_w(l):
        wpe, bpe = p['gl%d_pe' % l]
        wd = wpe[0, :G]
        ws = wpe[0, G:]
        wpt = jnp.concatenate(
            [wd[:, None], ws[:, None], jnp.zeros((G, 6), _f32)], axis=1)
        bp = jnp.concatenate([bpe, jnp.zeros((7,), _f32)])[None, :]
        return wpt, bp

    zero_wp = jnp.zeros((G, 8), _f32)
    zero_bp = jnp.zeros((1, 8), _f32)

    acc = _sweep_a(hp, ep, dsc, _bfw(wb), src, dst)
    wpt0, bp0 = prep_w(0)
    nf, prep = _fuse(acc, hv,
                     _bfw(p['gc_et'][0].T), p['gc_et'][1][None, :],
                     p['gc_gru_i'][0].T, p['gc_gru_i'][1][None, :],
                     p['gc_gru_h'][0].T, p['gc_gru_h'][1][None, :],
                     wpt0, bp0, True)
    for l in range(2):
        acc = _sweep_b(nf, prep[:, 0], prep[:, 1], src, dst)
        last = l == 1
        if last:
            wpt, bp = zero_wp, zero_bp
        else:
            wpt, bp = prep_w(l + 1)
        nf, prep = _fuse(acc, nf,
                         _bfw(p['gl%d_pn' % l][0].T), p['gl%d_pn' % l][1][None, :],
                         p['gl%d_gru_i' % l][0].T, p['gl%d_gru_i' % l][1][None, :],
                         p['gl%d_gru_h' % l][0].T, p['gl%d_gru_h' % l][1][None, :],
                         wpt, bp, not last)
    return nf


def kernel(h1, e1, h2, e2, params, edge_index1, graph_ids1, mask1,
           edge_index2, graph_ids2, mask2):
    p = params
    src1 = edge_index1[0].astype(jnp.int32)
    dst1 = edge_index1[1].astype(jnp.int32)
    src2 = edge_index2[0].astype(jnp.int32)
    dst2 = edge_index2[1].astype(jnp.int32)

    nf1 = _encode(p, h1, e1, src1, dst1)
    nf2 = _encode(p, h2, e2, src2, dst2)

    ro = {}
    for t in range(2):
        wcl, bcl = p['ro%d_cl' % t]
        ro['ro%d_cl' % t] = (wcl[0][:, None], bcl.reshape(1, 1))
        wpn, bpn = p['ro%d_pn' % t]
        ro['ro%d_pn' % t] = (wpn.T, bpn[None, :])
        ro['ro%d_gru' % t] = (p['ro%d_gru_i' % t][0].T,
                              p['ro%d_gru_i' % t][1][None, :],
                              p['ro%d_gru_h' % t][0].T,
                              p['ro%d_gru_h' % t][1][None, :])
    ro['ffn0'] = (p['ffn0'][0].T, p['ffn0'][1][None, :])
    ro['ffn1'] = (p['ffn1'][0].T, p['ffn1'][1][None, :])
    ro['ffn_out'] = (p['ffn_out'][0].T, p['ffn_out'][1][None, :])

    gid1 = graph_ids1.astype(jnp.int32)[:, None]
    gid2 = graph_ids2.astype(jnp.int32)[:, None]
    w1 = mask1.astype(_f32)[:, None]
    w2 = mask2.astype(_f32)[:, None]

    return _readout_ffn(ro, nf1, nf2, gid1, gid2, w1, w2)
